# baseline (speedup 1.0000x reference)
import functools

import jax
import jax.numpy as jnp
from jax import lax
from jax.experimental import pallas as pl
from jax.experimental.pallas import tpu as pltpu
from jax.experimental.pallas import tpu_sc as plsc

N = 10000
E = 320000
D = 128
G = 400
EP = 20
NC = 5
NQ = 10

NCORE = 2
NSUB = 16
NW = NCORE * NSUB
CH = 128
EPW = E // NW
NQT = 4
QCH = 20
CHUNKS = NQT * QCH
PPW = CHUNKS * CH - EPW
RPT = 640
R = NSUB * RPT

BN = 5000
NB = N // BN


def _sc_edge_agg(x, src3, dst3, zrows):
    @functools.partial(
        pl.kernel,
        out_type=jax.ShapeDtypeStruct((NCORE, R, D), jnp.float32),
        mesh=plsc.VectorSubcoreMesh(core_axis_name="c", subcore_axis_name="s"),
        scratch_types=[
            pltpu.VMEM((2, QCH, CH), jnp.int32),
            pltpu.VMEM((2, QCH, CH), jnp.int32),
            pltpu.VMEM((2, CH, D), jnp.float32),
            pltpu.VMEM_SHARED((R, D), jnp.float32),
            pltpu.SemaphoreType.DMA,
            pltpu.SemaphoreType.DMA,
            pltpu.SemaphoreType.DMA,
        ],
    )
    def body(x_hbm, src_hbm, dst_hbm, z_hbm, out_hbm, sidx_v, didx_v, rows_v,
             agg_sh, sem_i, sem_g, sem_s):
        c = lax.axis_index("c")
        s = lax.axis_index("s")
        wid = c * NSUB + s

        def stage(q, b):
            pltpu.async_copy(src_hbm.at[wid, q], sidx_v.at[b], sem_i)
            pltpu.async_copy(dst_hbm.at[wid, q], didx_v.at[b], sem_i)

        def stage_wait(q, b):
            pltpu.make_async_copy(src_hbm.at[wid, q], sidx_v.at[b],
                                  sem_i).wait()
            pltpu.make_async_copy(dst_hbm.at[wid, q], didx_v.at[b],
                                  sem_i).wait()

        stage(0, 0)
        pltpu.sync_copy(z_hbm, agg_sh.at[pl.ds(s * RPT, RPT)])
        plsc.subcore_barrier()
        stage_wait(0, 0)

        def drain(buf, sem_s):
            pltpu.make_async_copy(rows_v.at[buf],
                                  agg_sh.at[didx_v.at[0, 0]], sem_s).wait()

        for q in range(NQT):
            b = q % 2
            if q + 1 < NQT:
                stage(q + 1, 1 - b)

            def pair(j, carry):
                g0 = 2 * j
                g1 = g0 + 1
                dg0 = pltpu.async_copy(x_hbm.at[sidx_v.at[b, g0]],
                                       rows_v.at[0], sem_g)
                dg1 = pltpu.async_copy(x_hbm.at[sidx_v.at[b, g1]],
                                       rows_v.at[1], sem_g)
                dg0.wait()
                ds0 = pltpu.async_copy(rows_v.at[0],
                                       agg_sh.at[didx_v.at[b, g0]],
                                       sem_s, add=True)
                dg1.wait()
                ds1 = pltpu.async_copy(rows_v.at[1],
                                       agg_sh.at[didx_v.at[b, g1]],
                                       sem_s, add=True)
                ds0.wait()
                ds1.wait()
                return carry

            lax.fori_loop(0, QCH // 2, pair, 0)
            if q + 1 < NQT:
                stage_wait(q + 1, 1 - b)

        plsc.subcore_barrier()

        pltpu.sync_copy(agg_sh.at[pl.ds(s * RPT, RPT)],
                        out_hbm.at[c, pl.ds(s * RPT, RPT)])

    return body(x, src3, dst3, zrows)


def _tc_body(parts_ref, x_ref, wm_ref, ws_ref, gid_ref, pidx_ref,
             qidx_ref, out_ref, sums, counts):
    i = pl.program_id(0)

    @pl.when(i == 0)
    def _init():
        sums[...] = jnp.zeros_like(sums)
        counts[...] = jnp.zeros_like(counts)

    agg = parts_ref[0] + parts_ref[1]
    z = lax.dot_general(agg, wm_ref[...], (((1,), (0,)), ((), ())),
                        preferred_element_type=jnp.float32)
    z = z + lax.dot_general(x_ref[...], ws_ref[...], (((1,), (0,)), ((), ())),
                            preferred_element_type=jnp.float32)
    h = jnp.maximum(z, 0.0)

    gid = gid_ref[0]
    ohT = (lax.broadcasted_iota(jnp.int32, (G, BN), 0) == gid
           ).astype(jnp.float32)
    sums[...] += lax.dot_general(ohT, h, (((1,), (0,)), ((), ())),
                                 preferred_element_type=jnp.float32)
    counts[...] += jnp.sum(ohT, axis=1, keepdims=True)

    @pl.when(i == NB - 1)
    def _finish():
        ge = sums[...] / jnp.maximum(counts[...], 1.0)
        pidx = pidx_ref[...]
        qidx = qidx_ref[...]
        ohp = (pidx == lax.broadcasted_iota(jnp.int32, (EP * 8, G), 1)
               ).astype(jnp.float32)
        ohq = (qidx == lax.broadcasted_iota(jnp.int32, (EP * 16, G), 1)
               ).astype(jnp.float32)
        P = lax.dot_general(ohp, ge, (((1,), (0,)), ((), ())),
                            preferred_element_type=jnp.float32)
        Q = lax.dot_general(ohq, ge, (((1,), (0,)), ((), ())),
                            preferred_element_type=jnp.float32)
        pnrm = jnp.sqrt(jnp.sum(P * P, axis=1, keepdims=True))
        qnrm = jnp.sqrt(jnp.sum(Q * Q, axis=1, keepdims=True))
        pn = P / (pnrm + 1e-8)
        qn = Q / (qnrm + 1e-8)
        for e in range(EP):
            qe = lax.slice(qn, (e * 16, 0), (e * 16 + 16, D))
            pe = lax.slice(pn, (e * 8, 0), (e * 8 + 8, D))
            se = lax.dot_general(qe, pe, (((1,), (1,)), ((), ())),
                                 preferred_element_type=jnp.float32)
            out_ref[e] = lax.slice(se, (0, 0), (NQ, NC))


def _make_tc(interpret=False):
    return pl.pallas_call(
        _tc_body,
        grid=(NB,),
        in_specs=[
            pl.BlockSpec((NCORE, BN, D), lambda i: (0, i, 0)),
            pl.BlockSpec((BN, D), lambda i: (i, 0)),
            pl.BlockSpec((D, D), lambda i: (0, 0)),
            pl.BlockSpec((D, D), lambda i: (0, 0)),
            pl.BlockSpec((1, 1, BN), lambda i: (i, 0, 0)),
            pl.BlockSpec((EP * 8, 1), lambda i: (0, 0)),
            pl.BlockSpec((EP * 16, 1), lambda i: (0, 0)),
        ],
        out_specs=pl.BlockSpec((EP, NQ, NC), lambda i: (0, 0, 0)),
        out_shape=jax.ShapeDtypeStruct((EP, NQ, NC), jnp.float32),
        scratch_shapes=[
            pltpu.VMEM((G, D), jnp.float32),
            pltpu.VMEM((G, 1), jnp.float32),
        ],
        interpret=interpret,
    )


def kernel(x, W_msg, W_self, edge_index, graph_ids, prototype_indices,
           query_indices):
    src = edge_index[0]
    dst = edge_index[1]
    pad_iota = jnp.arange(PPW, dtype=jnp.int32)
    pad_src = jnp.broadcast_to(pad_iota[None, :], (NW, PPW))
    pad_dst = jnp.broadcast_to((N + pad_iota)[None, :], (NW, PPW))
    src3 = jnp.concatenate([src.reshape(NW, EPW), pad_src],
                           axis=1).reshape(NW, NQT, QCH, CH)
    dst3 = jnp.concatenate([dst.reshape(NW, EPW), pad_dst],
                           axis=1).reshape(NW, NQT, QCH, CH)
    zrows = jnp.zeros((RPT, D), jnp.float32)
    parts = _sc_edge_agg(x, src3, dst3, zrows)

    gid3 = graph_ids.reshape(NB, 1, BN)
    pidx = jnp.pad(prototype_indices.reshape(EP, NC),
                   ((0, 0), (0, 3))).reshape(EP * 8, 1)
    qidx = jnp.pad(query_indices.reshape(EP, NQ),
                   ((0, 0), (0, 6))).reshape(EP * 16, 1)
    return _make_tc()(parts, x, W_msg, W_self, gid3, pidx, qidx)


# speedup vs baseline: 1.0133x; 1.0133x over previous
import functools

import jax
import jax.numpy as jnp
from jax import lax
from jax.experimental import pallas as pl
from jax.experimental.pallas import tpu as pltpu
from jax.experimental.pallas import tpu_sc as plsc

N = 10000
E = 320000
D = 128
G = 400
EP = 20
NC = 5
NQ = 10

NCORE = 2
NSUB = 16
NW = NCORE * NSUB
CH = 128
EPW = E // NW
NQT = 4
QCH = 20
CHUNKS = NQT * QCH
PPW = CHUNKS * CH - EPW
RPT = 640
R = NSUB * RPT

BN = 2000
NB = N // BN


def _sc_edge_agg(x, src3, dst3, zrows):
    @functools.partial(
        pl.kernel,
        out_type=jax.ShapeDtypeStruct((NCORE, R, D), jnp.float32),
        mesh=plsc.VectorSubcoreMesh(core_axis_name="c", subcore_axis_name="s"),
        scratch_types=[
            pltpu.VMEM((2, QCH, CH), jnp.int32),
            pltpu.VMEM((2, QCH, CH), jnp.int32),
            pltpu.VMEM((2, CH, D), jnp.float32),
            pltpu.VMEM_SHARED((R, D), jnp.float32),
            pltpu.SemaphoreType.DMA,
            pltpu.SemaphoreType.DMA,
            pltpu.SemaphoreType.DMA,
        ],
    )
    def body(x_hbm, src_hbm, dst_hbm, z_hbm, out_hbm, sidx_v, didx_v, rows_v,
             agg_sh, sem_i, sem_g, sem_s):
        c = lax.axis_index("c")
        s = lax.axis_index("s")
        wid = c * NSUB + s

        def stage(q, b):
            pltpu.async_copy(src_hbm.at[wid, q], sidx_v.at[b], sem_i)
            pltpu.async_copy(dst_hbm.at[wid, q], didx_v.at[b], sem_i)

        def stage_wait(q, b):
            pltpu.make_async_copy(src_hbm.at[wid, q], sidx_v.at[b],
                                  sem_i).wait()
            pltpu.make_async_copy(dst_hbm.at[wid, q], didx_v.at[b],
                                  sem_i).wait()

        stage(0, 0)
        pltpu.sync_copy(z_hbm, agg_sh.at[pl.ds(s * RPT, RPT)])
        plsc.subcore_barrier()
        stage_wait(0, 0)

        for q in range(NQT):
            b = q % 2
            if q + 1 < NQT:
                stage(q + 1, 1 - b)

            def pair(j, carry):
                g0 = 2 * j
                g1 = g0 + 1
                dg0 = pltpu.async_copy(x_hbm.at[sidx_v.at[b, g0]],
                                       rows_v.at[0], sem_g)
                dg1 = pltpu.async_copy(x_hbm.at[sidx_v.at[b, g1]],
                                       rows_v.at[1], sem_g)
                dg0.wait()
                ds0 = pltpu.async_copy(rows_v.at[0],
                                       agg_sh.at[didx_v.at[b, g0]],
                                       sem_s, add=True)
                dg1.wait()
                ds1 = pltpu.async_copy(rows_v.at[1],
                                       agg_sh.at[didx_v.at[b, g1]],
                                       sem_s, add=True)
                ds0.wait()
                ds1.wait()
                return carry

            lax.fori_loop(0, QCH // 2, pair, 0)
            if q + 1 < NQT:
                stage_wait(q + 1, 1 - b)

        plsc.subcore_barrier()

        pltpu.sync_copy(agg_sh.at[pl.ds(s * RPT, RPT)],
                        out_hbm.at[c, pl.ds(s * RPT, RPT)])

    return body(x, src3, dst3, zrows)


def _tc_body(parts_ref, x_ref, wm_ref, ws_ref, gid_ref, pidx_ref,
             qidx_ref, out_ref, sums, counts):
    i = pl.program_id(0)

    @pl.when(i == 0)
    def _init():
        sums[...] = jnp.zeros_like(sums)
        counts[...] = jnp.zeros_like(counts)

    agg = parts_ref[0] + parts_ref[1]
    z = lax.dot_general(agg, wm_ref[...], (((1,), (0,)), ((), ())),
                        preferred_element_type=jnp.float32)
    z = z + lax.dot_general(x_ref[...], ws_ref[...], (((1,), (0,)), ((), ())),
                            preferred_element_type=jnp.float32)
    h = jnp.maximum(z, 0.0)

    gid = gid_ref[0]
    ohT = (lax.broadcasted_iota(jnp.int32, (G, BN), 0) == gid
           ).astype(jnp.float32)
    sums[...] += lax.dot_general(ohT, h, (((1,), (0,)), ((), ())),
                                 preferred_element_type=jnp.float32)
    counts[...] += jnp.sum(ohT, axis=1, keepdims=True)

    @pl.when(i == NB - 1)
    def _finish():
        ge = sums[...] / jnp.maximum(counts[...], 1.0)
        pidx = pidx_ref[...]
        qidx = qidx_ref[...]
        ohp = (pidx == lax.broadcasted_iota(jnp.int32, (EP * 8, G), 1)
               ).astype(jnp.float32)
        ohq = (qidx == lax.broadcasted_iota(jnp.int32, (EP * 16, G), 1)
               ).astype(jnp.float32)
        P = lax.dot_general(ohp, ge, (((1,), (0,)), ((), ())),
                            preferred_element_type=jnp.float32)
        Q = lax.dot_general(ohq, ge, (((1,), (0,)), ((), ())),
                            preferred_element_type=jnp.float32)
        pnrm = jnp.sqrt(jnp.sum(P * P, axis=1, keepdims=True))
        qnrm = jnp.sqrt(jnp.sum(Q * Q, axis=1, keepdims=True))
        pn = P / (pnrm + 1e-8)
        qn = Q / (qnrm + 1e-8)
        for e in range(EP):
            qe = lax.slice(qn, (e * 16, 0), (e * 16 + 16, D))
            pe = lax.slice(pn, (e * 8, 0), (e * 8 + 8, D))
            se = lax.dot_general(qe, pe, (((1,), (1,)), ((), ())),
                                 preferred_element_type=jnp.float32)
            out_ref[e] = lax.slice(se, (0, 0), (NQ, NC))


def _make_tc(interpret=False):
    return pl.pallas_call(
        _tc_body,
        grid=(NB,),
        in_specs=[
            pl.BlockSpec((NCORE, BN, D), lambda i: (0, i, 0)),
            pl.BlockSpec((BN, D), lambda i: (i, 0)),
            pl.BlockSpec((D, D), lambda i: (0, 0)),
            pl.BlockSpec((D, D), lambda i: (0, 0)),
            pl.BlockSpec((1, 1, BN), lambda i: (i, 0, 0)),
            pl.BlockSpec((EP * 8, 1), lambda i: (0, 0)),
            pl.BlockSpec((EP * 16, 1), lambda i: (0, 0)),
        ],
        out_specs=pl.BlockSpec((EP, NQ, NC), lambda i: (0, 0, 0)),
        out_shape=jax.ShapeDtypeStruct((EP, NQ, NC), jnp.float32),
        scratch_shapes=[
            pltpu.VMEM((G, D), jnp.float32),
            pltpu.VMEM((G, 1), jnp.float32),
        ],
        interpret=interpret,
    )


def kernel(x, W_msg, W_self, edge_index, graph_ids, prototype_indices,
           query_indices):
    src = edge_index[0]
    dst = edge_index[1]
    pad_iota = jnp.arange(PPW, dtype=jnp.int32)
    pad_src = jnp.broadcast_to(pad_iota[None, :], (NW, PPW))
    pad_dst = jnp.broadcast_to((N + pad_iota)[None, :], (NW, PPW))
    src3 = jnp.concatenate([src.reshape(NW, EPW), pad_src],
                           axis=1).reshape(NW, NQT, QCH, CH)
    dst3 = jnp.concatenate([dst.reshape(NW, EPW), pad_dst],
                           axis=1).reshape(NW, NQT, QCH, CH)
    zrows = jnp.zeros((RPT, D), jnp.float32)
    parts = _sc_edge_agg(x, src3, dst3, zrows)

    gid3 = graph_ids.reshape(NB, 1, BN)
    pidx = jnp.pad(prototype_indices.reshape(EP, NC),
                   ((0, 0), (0, 3))).reshape(EP * 8, 1)
    qidx = jnp.pad(query_indices.reshape(EP, NQ),
                   ((0, 0), (0, 6))).reshape(EP * 16, 1)
    return _make_tc()(parts, x, W_msg, W_self, gid3, pidx, qidx)
